# Initial kernel scaffold; baseline (speedup 1.0000x reference)
import jax
import jax.numpy as jnp
from jax.experimental import pallas as pl


def kernel(h_src, h_dst, edge_index, edge_weight, W1, b1, W2, b2):
    raise NotImplementedError("write your pallas kernel here")


# speedup vs baseline: 3.7733x; 3.7733x over previous
import jax
import jax.numpy as jnp
from jax import lax
from jax.experimental import pallas as pl
from jax.experimental.pallas import tpu as pltpu
from jax.experimental.pallas import tpu_sc as plsc

N = 10000
E = 320000
D = 128
DH = D // 2
NPAD = 10240

NC = 2
NS = 16
EPT = E // NS
C = 128
EPTP = ((EPT + C - 1) // C) * C
EPAD = EPTP * NS
NCHUNK = EPTP // C
RPT = NPAD // NS
ZCOPIES = RPT // C


def _fc1_body(x_ref, w_ref, b_ref, o_ref):
    y = jnp.dot(x_ref[...], w_ref[0], preferred_element_type=jnp.float32)
    o_ref[0] = jnp.maximum(y + b_ref[0], 0.0)


def _agg_body(hs_hbm, src_hbm, dst_hbm, w_hbm, out_f, out_w,
              src0_v, dst0_v, w0_v, src1_v, dst1_v, w1_v,
              rows0_v, rows1_v, half_v, wrow_v, zrow_v,
              acc_f, acc_w, sem_g0, sem_g1):
    cid = lax.axis_index("c")
    sid = lax.axis_index("s")

    def _zrow(r, _):
        for j in range(DH // 16):
            half_v[r, pl.ds(j * 16, 16)] = jnp.zeros((16,), jnp.float32)
        return 0
    lax.fori_loop(0, C, _zrow, 0)

    def _zroww(r, _):
        zrow_v[r, :] = jnp.zeros((16,), jnp.float32)
        return 0
    lax.fori_loop(0, RPT, _zroww, 0)

    for k in range(ZCOPIES):
        pltpu.sync_copy(half_v, acc_f.at[pl.ds(sid * RPT + k * C, C)])
    pltpu.sync_copy(zrow_v, acc_w.at[pl.ds(sid * RPT, RPT)])
    plsc.subcore_barrier()

    off16 = jnp.full((16,), cid * NPAD, jnp.int32)

    def _load_edges(c, srcv, dstv, wv):
        base = sid * EPTP + c * C
        pltpu.sync_copy(src_hbm.at[pl.ds(base, C)], srcv)
        pltpu.sync_copy(dst_hbm.at[pl.ds(base, C)], dstv)
        pltpu.sync_copy(w_hbm.at[pl.ds(base, C)], wv)
        for t in range(C // 16):
            srcv[pl.ds(t * 16, 16)] = srcv[pl.ds(t * 16, 16)] + off16

    def _drain(rowsv, sem):
        pltpu.make_async_copy(hs_hbm.at[pl.ds(0, C)], rowsv, sem).wait()

    def _compute(rowsv, dstv, wv):
        def _edge16(t, _):
            wvec = wv[pl.ds(t * 16, 16)]
            for l in range(16):
                r = t * 16 + l
                w16 = jnp.full((16,), wvec[l], jnp.float32)
                for j in range(DH // 16):
                    x = rowsv[r, pl.ds(j * 16, 16)]
                    half_v[r, pl.ds(j * 16, 16)] = x * w16
                wrow_v[r, :] = w16
            return 0
        lax.fori_loop(0, C // 16, _edge16, 0)
        pltpu.sync_copy(half_v, acc_f.at[dstv], add=True)
        pltpu.sync_copy(wrow_v, acc_w.at[dstv], add=True)

    _load_edges(0, src0_v, dst0_v, w0_v)
    pltpu.async_copy(hs_hbm.at[src0_v], rows0_v, sem_g0)

    def _pair(i, _):
        c0 = 2 * i
        _load_edges(c0 + 1, src1_v, dst1_v, w1_v)
        pltpu.async_copy(hs_hbm.at[src1_v], rows1_v, sem_g1)
        _drain(rows0_v, sem_g0)
        _compute(rows0_v, dst0_v, w0_v)
        _load_edges(c0 + 2, src0_v, dst0_v, w0_v)
        pltpu.async_copy(hs_hbm.at[src0_v], rows0_v, sem_g0)
        _drain(rows1_v, sem_g1)
        _compute(rows1_v, dst1_v, w1_v)
        return 0
    lax.fori_loop(0, (NCHUNK - 1) // 2, _pair, 0)
    _drain(rows0_v, sem_g0)
    _compute(rows0_v, dst0_v, w0_v)

    plsc.subcore_barrier()

    for k in range(ZCOPIES):
        b = sid * RPT + k * C
        pltpu.sync_copy(acc_f.at[pl.ds(b, C)], half_v)
        pltpu.sync_copy(half_v, out_f.at[cid, pl.ds(b, C)])
    pltpu.sync_copy(acc_w.at[pl.ds(sid * RPT, RPT)], zrow_v)
    pltpu.sync_copy(zrow_v, out_w.at[cid, pl.ds(sid * RPT, RPT)])


def _fc2_body(accf_ref, accw_ref, hd_ref, w2_ref, b2_ref, o_ref, ss_ref):
    i = pl.program_id(0)
    vs = jnp.concatenate([accf_ref[0], accf_ref[1]], axis=-1)
    ws = accw_ref[0, :, 0:1]
    ws = jnp.maximum(ws, 1.0)
    nv = vs / ws
    y = jnp.dot(nv, w2_ref[0:D, :], preferred_element_type=jnp.float32)
    y = y + jnp.dot(hd_ref[...], w2_ref[D:2 * D, :],
                    preferred_element_type=jnp.float32)
    y = jnp.maximum(y + b2_ref[...], 0.0)
    o_ref[...] = y
    br = o_ref.shape[0]
    row = i * br + lax.broadcasted_iota(jnp.int32, (br, 1), 0)
    sq = jnp.where(row < N, y * y, 0.0)

    @pl.when(i == 0)
    def _():
        ss_ref[...] = jnp.zeros((1, 1), jnp.float32)
    ss_ref[...] += jnp.sum(sq).reshape(1, 1)


def _norm_body(x_ref, ss_ref, o_ref):
    o_ref[...] = x_ref[...] * lax.rsqrt(ss_ref[0, 0])


def kernel(h_src, h_dst, edge_index, edge_weight, W1, b1, W2, b2):
    f32 = jnp.float32
    h_src_p = jnp.pad(h_src, ((0, NPAD - N), (0, 0)))
    h_dst_p = jnp.pad(h_dst, ((0, NPAD - N), (0, 0)))
    src = jnp.pad(edge_index[0].astype(jnp.int32), (0, EPAD - E))
    dst = jnp.pad(edge_index[1].astype(jnp.int32), (0, EPAD - E))
    b1r = b1.reshape(1, D).astype(f32)
    b2r = b2.reshape(1, D).astype(f32)

    W1s = W1.astype(f32).reshape(D, NC, DH).transpose(1, 0, 2)
    b1s = b1.astype(f32).reshape(NC, 1, DH)
    BR1 = 512
    hs2 = pl.pallas_call(
        _fc1_body,
        grid=(NC, NPAD // BR1),
        in_specs=[
            pl.BlockSpec((BR1, D), lambda h, i: (i, 0)),
            pl.BlockSpec((1, D, DH), lambda h, i: (h, 0, 0)),
            pl.BlockSpec((1, 1, DH), lambda h, i: (h, 0, 0)),
        ],
        out_specs=pl.BlockSpec((1, BR1, DH), lambda h, i: (h, i, 0)),
        out_shape=jax.ShapeDtypeStruct((NC, NPAD, DH), f32),
    )(h_src_p, W1s, b1s)
    hs = hs2.reshape(NC * NPAD, DH)

    mesh = plsc.VectorSubcoreMesh(core_axis_name="c", subcore_axis_name="s")
    agg = pl.kernel(
        _agg_body,
        out_type=(
            jax.ShapeDtypeStruct((NC, NPAD, DH), f32),
            jax.ShapeDtypeStruct((NC, NPAD, 16), f32),
        ),
        mesh=mesh,
        compiler_params=pltpu.CompilerParams(use_tc_tiling_on_sc=False),
        scratch_types=[
            pltpu.VMEM((C,), jnp.int32),
            pltpu.VMEM((C,), jnp.int32),
            pltpu.VMEM((C,), f32),
            pltpu.VMEM((C,), jnp.int32),
            pltpu.VMEM((C,), jnp.int32),
            pltpu.VMEM((C,), f32),
            pltpu.VMEM((C, DH), f32),
            pltpu.VMEM((C, DH), f32),
            pltpu.VMEM((C, DH), f32),
            pltpu.VMEM((C, 16), f32),
            pltpu.VMEM((RPT, 16), f32),
            pltpu.VMEM_SHARED((NPAD, DH), f32),
            pltpu.VMEM_SHARED((NPAD, 16), f32),
            pltpu.SemaphoreType.DMA,
            pltpu.SemaphoreType.DMA,
        ],
    )
    ew = jnp.pad(edge_weight.astype(f32), (0, EPAD - E))
    acc_f, acc_w = agg(hs, src, dst, ew)

    BR2 = 256
    new_pre, sumsq = pl.pallas_call(
        _fc2_body,
        grid=(NPAD // BR2,),
        in_specs=[
            pl.BlockSpec((NC, BR2, DH), lambda i: (0, i, 0)),
            pl.BlockSpec((1, BR2, 16), lambda i: (0, i, 0)),
            pl.BlockSpec((BR2, D), lambda i: (i, 0)),
            pl.BlockSpec((2 * D, D), lambda i: (0, 0)),
            pl.BlockSpec((1, D), lambda i: (0, 0)),
        ],
        out_specs=(
            pl.BlockSpec((BR2, D), lambda i: (i, 0)),
            pl.BlockSpec((1, 1), lambda i: (0, 0)),
        ),
        out_shape=(
            jax.ShapeDtypeStruct((NPAD, D), f32),
            jax.ShapeDtypeStruct((1, 1), f32),
        ),
    )(acc_f, acc_w, h_dst_p, W2, b2r)

    BR3 = 512
    new = pl.pallas_call(
        _norm_body,
        grid=(NPAD // BR3,),
        in_specs=[
            pl.BlockSpec((BR3, D), lambda i: (i, 0)),
            pl.BlockSpec((1, 1), lambda i: (0, 0)),
        ],
        out_specs=pl.BlockSpec((BR3, D), lambda i: (i, 0)),
        out_shape=jax.ShapeDtypeStruct((NPAD, D), f32),
    )(new_pre, sumsq)

    return new[:N]
